# baseline (speedup 1.0000x reference)
import jax
import jax.numpy as jnp
from jax.experimental import pallas as pl
from jax.experimental.pallas import tpu as pltpu

_E = 16
_D = 2048
_TM = 512
_NBUF = 6


def _block_compute(x, wt, b2, onehot_ref, plog_ref, row0):
    logits = jnp.dot(x, wt, preferred_element_type=jnp.float32) + b2
    m = jnp.max(logits, axis=-1, keepdims=True)
    e = jnp.exp(logits - m)
    s = jnp.sum(e, axis=-1, keepdims=True)
    p = e / s
    pmax = jnp.max(p, axis=-1, keepdims=True)
    ii = jax.lax.broadcasted_iota(jnp.int32, p.shape, 1)
    idx = jnp.min(jnp.where(p == pmax, ii, _E), axis=-1, keepdims=True)
    onehot_ref[row0:row0 + _TM, :] = (ii == idx).astype(jnp.int32)
    plog_ref[row0:row0 + _TM, :] = pmax


def _router(x_hbm, wt_ref, b_ref, onehot_ref, plog_ref, *scratch):
    bufs = scratch[:_NBUF]
    sems = scratch[_NBUF:]
    M = x_hbm.shape[0]
    nb = M // _TM
    wt = wt_ref[...]
    b2 = b_ref[...]

    def _copy(k, slot):
        return pltpu.make_async_copy(
            x_hbm.at[pl.ds(k * _TM, _TM), :], bufs[slot], sems[slot]
        )

    for s in range(_NBUF):
        _copy(s, s).start()
    for k in range(nb):
        slot = k % _NBUF
        _copy(k, slot).wait()
        _block_compute(bufs[slot][...], wt, b2, onehot_ref, plog_ref, k * _TM)
        if k + _NBUF < nb:
            _copy(k + _NBUF, slot).start()


def kernel(hidden_states, W, b):
    B, S, D = hidden_states.shape
    M = B * S
    x = hidden_states.reshape(M, D)
    wt = W.T
    b2 = b.reshape(1, _E)
    onehot, plog = pl.pallas_call(
        _router,
        in_specs=[
            pl.BlockSpec(memory_space=pltpu.HBM),
            pl.BlockSpec(memory_space=pltpu.VMEM),
            pl.BlockSpec(memory_space=pltpu.VMEM),
        ],
        out_specs=[
            pl.BlockSpec(memory_space=pltpu.VMEM),
            pl.BlockSpec(memory_space=pltpu.VMEM),
        ],
        out_shape=[
            jax.ShapeDtypeStruct((M, _E), jnp.int32),
            jax.ShapeDtypeStruct((M, 1), jnp.float32),
        ],
        scratch_shapes=(
            [pltpu.VMEM((_TM, _D), jnp.float32) for _ in range(_NBUF)]
            + [pltpu.SemaphoreType.DMA for _ in range(_NBUF)]
        ),
    )(x, wt, b2)
    return (onehot.reshape(B, S, 1, _E), plog.reshape(B, S, 1))


# speedup vs baseline: 1.2661x; 1.2661x over previous
import jax
import jax.numpy as jnp
from jax.experimental import pallas as pl
from jax.experimental.pallas import tpu as pltpu

_E = 16
_D = 2048
_TM = 1024


def _router_block(x_ref, wt_ref, b_ref, onehot_ref, plog_ref):
    x = x_ref[...]
    logits = jnp.dot(x, wt_ref[...], preferred_element_type=jnp.float32)
    logits = logits + b_ref[...]
    m = jnp.max(logits, axis=-1, keepdims=True)
    e = jnp.exp(logits - m)
    s = jnp.sum(e, axis=-1, keepdims=True)
    p = e / s
    pmax = jnp.max(p, axis=-1, keepdims=True)
    ii = jax.lax.broadcasted_iota(jnp.int32, p.shape, 1)
    idx = jnp.min(jnp.where(p == pmax, ii, _E), axis=-1, keepdims=True)
    onehot_ref[...] = (ii == idx).astype(jnp.int32)
    plog_ref[...] = pmax


def kernel(hidden_states, W, b):
    B, S, D = hidden_states.shape
    M = B * S
    x = hidden_states.reshape(M, D)
    wt = W.T
    b2 = b.reshape(1, _E)
    grid = (M // _TM,)
    onehot, plog = pl.pallas_call(
        _router_block,
        grid=grid,
        in_specs=[
            pl.BlockSpec((_TM, D), lambda i: (i, 0)),
            pl.BlockSpec((D, _E), lambda i: (0, 0)),
            pl.BlockSpec((1, _E), lambda i: (0, 0)),
        ],
        out_specs=[
            pl.BlockSpec((_TM, _E), lambda i: (i, 0)),
            pl.BlockSpec((_TM, 1), lambda i: (i, 0)),
        ],
        out_shape=[
            jax.ShapeDtypeStruct((M, _E), jnp.int32),
            jax.ShapeDtypeStruct((M, 1), jnp.float32),
        ],
        compiler_params=pltpu.CompilerParams(
            dimension_semantics=("parallel",),
        ),
    )(x, wt, b2)
    return (onehot.reshape(B, S, 1, _E), plog.reshape(B, S, 1))
